# Initial kernel scaffold; baseline (speedup 1.0000x reference)
import jax
import jax.numpy as jnp
from jax.experimental import pallas as pl


def kernel(features, fp_w1, fp_b1, fp_w2, fp_b2, conv_w, conv_b, codebook, gru_w_ih, gru_w_hh, gru_b_ih, gru_b_hh, word_codebook, align_w, align_b):
    raise NotImplementedError("write your pallas kernel here")


# speedup vs baseline: 2.9976x; 2.9976x over previous
import functools

import jax
import jax.numpy as jnp
from jax.experimental import pallas as pl
from jax.experimental.pallas import tpu as pltpu

B, T, D_IN = 16, 512, 1024
CD, CS, WS = 256, 1024, 128
TW = T // 2
H1 = 512
G3 = 3 * CD

_HI = jax.lax.Precision.HIGHEST
_INTERPRET = False


def _dot(a, b):
    return jax.lax.dot_general(a, b, (((1,), (0,)), ((), ())),
                               preferred_element_type=jnp.float32)


def _dotx(a, b):
    return jax.lax.dot_general(a, b, (((1,), (0,)), ((), ())),
                               precision=_HI, preferred_element_type=jnp.float32)


def _k1_body(feat_ref, w1t_ref, b1_ref, w2t_ref, b2_ref,
             cw0_ref, cw1_ref, cw2_ref, cb_ref, cbt_ref, cbz_ref,
             wcb_ref, wcbt_ref, wihT_ref, bih_ref, awt_ref, ab_ref,
             idx_ref, widx_ref, gi_ref, feats_ref, commit_ref, align_ref,
             h2pad_ref):
    b = pl.program_id(0)

    @pl.when(b == 0)
    def _init():
        h2pad_ref[0:8, :] = jnp.zeros((8, CD), jnp.float32)
        h2pad_ref[520:528, :] = jnp.zeros((8, CD), jnp.float32)
        commit_ref[...] = jnp.zeros((1, 1), jnp.float32)
        align_ref[...] = jnp.zeros((1, 1), jnp.float32)

    x = feat_ref[0]
    h1 = jnp.maximum(_dot(x, w1t_ref[...]) + b1_ref[...], 0.0)
    h2 = jnp.maximum(_dot(h1, w2t_ref[...]) + b2_ref[...], 0.0)
    h2pad_ref[8:520, :] = h2

    down = h2pad_ref[7:519, :]
    mid = h2pad_ref[8:520, :]
    up = h2pad_ref[9:521, :]
    feats = (_dot(down, cw0_ref[...]) + _dot(mid, cw1_ref[...])
             + _dot(up, cw2_ref[...]) + cbz_ref[...])
    feats_ref[0] = feats

    fsq = jnp.sum(feats * feats, axis=1, keepdims=True)
    dots = _dot(feats, cbt_ref[...])
    csq = jnp.sum(cbt_ref[...] * cbt_ref[...], axis=0, keepdims=True)
    d2 = (fsq - 2.0 * dots) + csq
    mind = jnp.min(d2, axis=1, keepdims=True)
    it = jax.lax.broadcasted_iota(jnp.int32, (T, CS), 1)
    idx = jnp.min(jnp.where(d2 == mind, it, CS), axis=1, keepdims=True)
    idx_ref[0] = idx
    commit_ref[...] += jnp.sum(mind, keepdims=True)

    onehot = jnp.where(it == idx, 1.0, 0.0)
    quant = _dotx(onehot, cb_ref[...])
    gi_ref[0] = _dot(quant, wihT_ref[...]) + bih_ref[...]

    iw = jax.lax.broadcasted_iota(jnp.int32, (TW, T), 0)
    itt = jax.lax.broadcasted_iota(jnp.int32, (TW, T), 1)
    pmat = jnp.where((itt // 2) == iw, 0.5, 0.0)
    wm = _dotx(pmat, quant)
    wsq = jnp.sum(wm * wm, axis=1, keepdims=True)
    wdots = _dot(wm, wcbt_ref[...])
    wcsq = jnp.sum(wcbt_ref[...] * wcbt_ref[...], axis=0, keepdims=True)
    d2w = (wsq - 2.0 * wdots) + wcsq
    wmind = jnp.min(d2w, axis=1, keepdims=True)
    itw = jax.lax.broadcasted_iota(jnp.int32, (TW, WS), 1)
    widx = jnp.min(jnp.where(d2w == wmind, itw, WS), axis=1, keepdims=True)
    widx_ref[0] = widx
    counts = jnp.sum(jnp.where(itw == widx, 1.0, 0.0), axis=0, keepdims=True)
    pdiff = _dot(wcb_ref[...], awt_ref[...]) + ab_ref[...] - wcb_ref[...]
    pe = jnp.sum(pdiff * pdiff, axis=1, keepdims=True)
    align_ref[...] += _dotx(counts, pe)


def _k2_body(gi_ref, feats_ref, whht_ref, bhh_ref, ctx_ref):
    def step(t, carry):
        h, acc = carry
        gx = gi_ref[t]
        gh = _dot(h, whht_ref[...]) + bhh_ref[...]
        r = jax.nn.sigmoid(gx[:, 0:CD] + gh[:, 0:CD])
        z = jax.nn.sigmoid(gx[:, CD:2 * CD] + gh[:, CD:2 * CD])
        n = jnp.tanh(gx[:, 2 * CD:] + r * gh[:, 2 * CD:])
        h_new = (1.0 - z) * n + z * h
        d = h_new - feats_ref[t + 1]
        return h_new, acc + jnp.sum(d * d, keepdims=True)

    h0 = jnp.zeros((B, CD), jnp.float32)
    _, acc = jax.lax.fori_loop(0, T - 1, step, (h0, jnp.zeros((1, 1), jnp.float32)))
    ctx_ref[...] = acc


def kernel(features, fp_w1, fp_b1, fp_w2, fp_b2, conv_w, conv_b, codebook,
           gru_w_ih, gru_w_hh, gru_b_ih, gru_b_hh, word_codebook, align_w,
           align_b):
    f32 = jnp.float32
    w1t = fp_w1.T
    b1 = fp_b1.reshape(1, H1)
    w2t = fp_w2.T
    b2 = fp_b2.reshape(1, CD)
    cw0 = conv_w[:, :, 0].T
    cw1 = conv_w[:, :, 1].T
    cw2 = conv_w[:, :, 2].T
    cbz = conv_b.reshape(1, CD)
    cbt = codebook.T
    wcbt = word_codebook.T
    wihT = gru_w_ih.T
    bih = gru_b_ih.reshape(1, G3)
    whht = gru_w_hh.T
    bhh = gru_b_hh.reshape(1, G3)
    awt = align_w.T
    ab = align_b.reshape(1, CD)

    grid1 = (B,)
    out1 = (
        jax.ShapeDtypeStruct((B, T, 1), jnp.int32),
        jax.ShapeDtypeStruct((B, TW, 1), jnp.int32),
        jax.ShapeDtypeStruct((B, T, G3), f32),
        jax.ShapeDtypeStruct((B, T, CD), f32),
        jax.ShapeDtypeStruct((1, 1), f32),
        jax.ShapeDtypeStruct((1, 1), f32),
    )
    in_specs1 = [
        pl.BlockSpec((1, T, D_IN), lambda b: (b, 0, 0)),
        pl.BlockSpec((D_IN, H1), lambda b: (0, 0)),
        pl.BlockSpec((1, H1), lambda b: (0, 0)),
        pl.BlockSpec((H1, CD), lambda b: (0, 0)),
        pl.BlockSpec((1, CD), lambda b: (0, 0)),
        pl.BlockSpec((CD, CD), lambda b: (0, 0)),
        pl.BlockSpec((CD, CD), lambda b: (0, 0)),
        pl.BlockSpec((CD, CD), lambda b: (0, 0)),
        pl.BlockSpec((CS, CD), lambda b: (0, 0)),
        pl.BlockSpec((CD, CS), lambda b: (0, 0)),
        pl.BlockSpec((1, CD), lambda b: (0, 0)),
        pl.BlockSpec((WS, CD), lambda b: (0, 0)),
        pl.BlockSpec((CD, WS), lambda b: (0, 0)),
        pl.BlockSpec((CD, G3), lambda b: (0, 0)),
        pl.BlockSpec((1, G3), lambda b: (0, 0)),
        pl.BlockSpec((CD, CD), lambda b: (0, 0)),
        pl.BlockSpec((1, CD), lambda b: (0, 0)),
    ]
    out_specs1 = (
        pl.BlockSpec((1, T, 1), lambda b: (b, 0, 0)),
        pl.BlockSpec((1, TW, 1), lambda b: (b, 0, 0)),
        pl.BlockSpec((1, T, G3), lambda b: (b, 0, 0)),
        pl.BlockSpec((1, T, CD), lambda b: (b, 0, 0)),
        pl.BlockSpec((1, 1), lambda b: (0, 0)),
        pl.BlockSpec((1, 1), lambda b: (0, 0)),
    )
    idx3, widx3, gi, feats, commit_s, align_s = pl.pallas_call(
        _k1_body,
        grid=grid1,
        in_specs=in_specs1,
        out_specs=out_specs1,
        out_shape=out1,
        scratch_shapes=[pltpu.VMEM((528, CD), f32)],
        interpret=_INTERPRET,
    )(features, w1t, b1, w2t, b2, cw0, cw1, cw2, codebook, cbt, cbz,
      word_codebook, wcbt, wihT, bih, awt, ab)

    gi_t = jnp.transpose(gi, (1, 0, 2))
    feats_t = jnp.transpose(feats, (1, 0, 2))
    ctx_s = pl.pallas_call(
        _k2_body,
        in_specs=[
            pl.BlockSpec((T, B, G3), lambda: (0, 0, 0)),
            pl.BlockSpec((T, B, CD), lambda: (0, 0, 0)),
            pl.BlockSpec((CD, G3), lambda: (0, 0)),
            pl.BlockSpec((1, G3), lambda: (0, 0)),
        ],
        out_specs=pl.BlockSpec((1, 1), lambda: (0, 0)),
        out_shape=jax.ShapeDtypeStruct((1, 1), f32),
        interpret=_INTERPRET,
    )(gi_t, feats_t, whht, bhh)

    token_indices = idx3[:, :, 0]
    widx = widx3[:, :, 0]
    commitment_loss = commit_s[0, 0] / (B * T * CD)
    codebook_loss = commitment_loss
    context_loss = ctx_s[0, 0] / (B * (T - 1) * CD)
    vq_loss = commitment_loss + 0.25 * codebook_loss + 0.1 * context_loss
    alignment_loss = align_s[0, 0] / (B * TW * CD)
    total_loss = vq_loss + alignment_loss * 0.1
    return (widx, token_indices, commitment_loss, codebook_loss, context_loss,
            vq_loss, alignment_loss, total_loss)


# speedup vs baseline: 3.9800x; 1.3277x over previous
import functools

import jax
import jax.numpy as jnp
from jax.experimental import pallas as pl
from jax.experimental.pallas import tpu as pltpu

B, T, D_IN = 16, 512, 1024
CD, CS, WS = 256, 1024, 128
TW = T // 2
H1 = 512
G3 = 3 * CD

_HI = jax.lax.Precision.HIGHEST
_INTERPRET = False


def _dot(a, b):
    return jax.lax.dot_general(a, b, (((1,), (0,)), ((), ())),
                               preferred_element_type=jnp.float32)


def _dotx(a, b):
    return jax.lax.dot_general(a, b, (((1,), (0,)), ((), ())),
                               precision=_HI, preferred_element_type=jnp.float32)


def _k1_body(feat_ref, w1t_ref, b1_ref, w2t_ref, b2_ref,
             cw0_ref, cw1_ref, cw2_ref, cb_ref, cbt_ref, cbz_ref,
             wcb_ref, wcbt_ref, wihT_ref, bih_ref, awt_ref, ab_ref,
             idx_ref, widx_ref, gi_ref, feats_ref, commit_ref, align_ref,
             h2pad_ref):
    h2pad_ref[0:8, :] = jnp.zeros((8, CD), jnp.float32)
    h2pad_ref[520:528, :] = jnp.zeros((8, CD), jnp.float32)

    x = feat_ref[0]
    h1 = jnp.maximum(_dot(x, w1t_ref[...]) + b1_ref[...], 0.0)
    h2 = jnp.maximum(_dot(h1, w2t_ref[...]) + b2_ref[...], 0.0)
    h2pad_ref[8:520, :] = h2

    down = h2pad_ref[7:519, :]
    mid = h2pad_ref[8:520, :]
    up = h2pad_ref[9:521, :]
    feats = (_dot(down, cw0_ref[...]) + _dot(mid, cw1_ref[...])
             + _dot(up, cw2_ref[...]) + cbz_ref[...])
    feats_ref[0] = feats

    fsq = jnp.sum(feats * feats, axis=1, keepdims=True)
    dots = _dot(feats, cbt_ref[...])
    csq = jnp.sum(cbt_ref[...] * cbt_ref[...], axis=0, keepdims=True)
    d2 = (fsq - 2.0 * dots) + csq
    mind = jnp.min(d2, axis=1, keepdims=True)
    it = jax.lax.broadcasted_iota(jnp.int32, (T, CS), 1)
    idx = jnp.min(jnp.where(d2 == mind, it, CS), axis=1, keepdims=True)
    idx_ref[0] = idx
    commit_ref[0] = jnp.sum(mind, keepdims=True)

    onehot = jnp.where(it == idx, 1.0, 0.0)
    quant = _dot(onehot, cb_ref[...])
    gi_ref[0] = _dot(quant, wihT_ref[...]) + bih_ref[...]

    iw = jax.lax.broadcasted_iota(jnp.int32, (TW, T), 0)
    itt = jax.lax.broadcasted_iota(jnp.int32, (TW, T), 1)
    pmat = jnp.where((itt // 2) == iw, 0.5, 0.0)
    poh = _dot(pmat, onehot)
    wm = _dotx(poh, cb_ref[...])
    wsq = jnp.sum(wm * wm, axis=1, keepdims=True)
    wdots = _dot(wm, wcbt_ref[...])
    wcsq = jnp.sum(wcbt_ref[...] * wcbt_ref[...], axis=0, keepdims=True)
    d2w = (wsq - 2.0 * wdots) + wcsq
    wmind = jnp.min(d2w, axis=1, keepdims=True)
    itw = jax.lax.broadcasted_iota(jnp.int32, (TW, WS), 1)
    widx = jnp.min(jnp.where(d2w == wmind, itw, WS), axis=1, keepdims=True)
    widx_ref[0] = widx
    counts = jnp.sum(jnp.where(itw == widx, 1.0, 0.0), axis=0, keepdims=True)
    pdiff = _dot(wcb_ref[...], awt_ref[...]) + ab_ref[...] - wcb_ref[...]
    pe = jnp.sum(pdiff * pdiff, axis=1, keepdims=True)
    align_ref[0] = _dotx(counts, pe)


def _k2_body(gi_ref, feats_ref, whht_ref, ctx_ref):
    def step(t, carry):
        h, acc = carry
        d = h - feats_ref[t]
        acc = acc + jnp.where(t > 0, d * d, jnp.zeros((B, CD), jnp.float32))
        gx = gi_ref[t]
        gh = _dot(h, whht_ref[...])
        r = jax.nn.sigmoid(gx[:, 0:CD] + gh[:, 0:CD])
        z = jax.nn.sigmoid(gx[:, CD:2 * CD] + gh[:, CD:2 * CD])
        n = jnp.tanh(gx[:, 2 * CD:] + r * gh[:, 2 * CD:])
        h_new = (1.0 - z) * n + z * h
        return h_new, acc

    h0 = jnp.zeros((B, CD), jnp.float32)
    h, acc = jax.lax.fori_loop(0, T - 1, step,
                               (h0, jnp.zeros((B, CD), jnp.float32)))
    d = h - feats_ref[T - 1]
    ctx_ref[...] = jnp.sum(acc + d * d, keepdims=True)


def kernel(features, fp_w1, fp_b1, fp_w2, fp_b2, conv_w, conv_b, codebook,
           gru_w_ih, gru_w_hh, gru_b_ih, gru_b_hh, word_codebook, align_w,
           align_b):
    f32 = jnp.float32
    w1t = fp_w1.T
    b1 = fp_b1.reshape(1, H1)
    w2t = fp_w2.T
    b2 = fp_b2.reshape(1, CD)
    cw0 = conv_w[:, :, 0].T
    cw1 = conv_w[:, :, 1].T
    cw2 = conv_w[:, :, 2].T
    cbz = conv_b.reshape(1, CD)
    cbt = codebook.T
    wcbt = word_codebook.T
    wihT = gru_w_ih.T
    whht = gru_w_hh.T
    bih = (gru_b_ih + gru_b_hh).reshape(1, G3)
    awt = align_w.T
    ab = align_b.reshape(1, CD)

    grid1 = (B,)
    out1 = (
        jax.ShapeDtypeStruct((B, T, 1), jnp.int32),
        jax.ShapeDtypeStruct((B, TW, 1), jnp.int32),
        jax.ShapeDtypeStruct((B, T, G3), f32),
        jax.ShapeDtypeStruct((B, T, CD), f32),
        jax.ShapeDtypeStruct((B, 1, 1), f32),
        jax.ShapeDtypeStruct((B, 1, 1), f32),
    )
    in_specs1 = [
        pl.BlockSpec((1, T, D_IN), lambda b: (b, 0, 0)),
        pl.BlockSpec((D_IN, H1), lambda b: (0, 0)),
        pl.BlockSpec((1, H1), lambda b: (0, 0)),
        pl.BlockSpec((H1, CD), lambda b: (0, 0)),
        pl.BlockSpec((1, CD), lambda b: (0, 0)),
        pl.BlockSpec((CD, CD), lambda b: (0, 0)),
        pl.BlockSpec((CD, CD), lambda b: (0, 0)),
        pl.BlockSpec((CD, CD), lambda b: (0, 0)),
        pl.BlockSpec((CS, CD), lambda b: (0, 0)),
        pl.BlockSpec((CD, CS), lambda b: (0, 0)),
        pl.BlockSpec((1, CD), lambda b: (0, 0)),
        pl.BlockSpec((WS, CD), lambda b: (0, 0)),
        pl.BlockSpec((CD, WS), lambda b: (0, 0)),
        pl.BlockSpec((CD, G3), lambda b: (0, 0)),
        pl.BlockSpec((1, G3), lambda b: (0, 0)),
        pl.BlockSpec((CD, CD), lambda b: (0, 0)),
        pl.BlockSpec((1, CD), lambda b: (0, 0)),
    ]
    out_specs1 = (
        pl.BlockSpec((1, T, 1), lambda b: (b, 0, 0)),
        pl.BlockSpec((1, TW, 1), lambda b: (b, 0, 0)),
        pl.BlockSpec((1, T, G3), lambda b: (b, 0, 0)),
        pl.BlockSpec((1, T, CD), lambda b: (b, 0, 0)),
        pl.BlockSpec((1, 1, 1), lambda b: (b, 0, 0)),
        pl.BlockSpec((1, 1, 1), lambda b: (b, 0, 0)),
    )
    idx3, widx3, gi, feats, commit_s, align_s = pl.pallas_call(
        _k1_body,
        grid=grid1,
        in_specs=in_specs1,
        out_specs=out_specs1,
        out_shape=out1,
        scratch_shapes=[pltpu.VMEM((528, CD), f32)],
        compiler_params=pltpu.CompilerParams(
            dimension_semantics=("parallel",)),
        interpret=_INTERPRET,
    )(features, w1t, b1, w2t, b2, cw0, cw1, cw2, codebook, cbt, cbz,
      word_codebook, wcbt, wihT, bih, awt, ab)

    gi_t = jnp.transpose(gi, (1, 0, 2))
    feats_t = jnp.transpose(feats, (1, 0, 2))
    ctx_s = pl.pallas_call(
        _k2_body,
        in_specs=[
            pl.BlockSpec((T, B, G3), lambda: (0, 0, 0)),
            pl.BlockSpec((T, B, CD), lambda: (0, 0, 0)),
            pl.BlockSpec((CD, G3), lambda: (0, 0)),
        ],
        out_specs=pl.BlockSpec((1, 1), lambda: (0, 0)),
        out_shape=jax.ShapeDtypeStruct((1, 1), f32),
        interpret=_INTERPRET,
    )(gi_t, feats_t, whht)

    token_indices = idx3[:, :, 0]
    widx = widx3[:, :, 0]
    commitment_loss = jnp.sum(commit_s) / (B * T * CD)
    codebook_loss = commitment_loss
    context_loss = ctx_s[0, 0] / (B * (T - 1) * CD)
    vq_loss = commitment_loss + 0.25 * codebook_loss + 0.1 * context_loss
    alignment_loss = jnp.sum(align_s) / (B * TW * CD)
    total_loss = vq_loss + alignment_loss * 0.1
    return (widx, token_indices, commitment_loss, codebook_loss, context_loss,
            vq_loss, alignment_loss, total_loss)
